# baseline (speedup 1.0000x reference)
import functools

import jax
import jax.numpy as jnp
from jax import lax
from jax.experimental import pallas as pl
from jax.experimental.pallas import tpu as pltpu
from jax.experimental.pallas import tpu_sc as plsc

NC = 2
NS = 16
NW = NC * NS
LANES = 16

BATCH = 16384
FEAT = 64
B_PER_W = BATCH // NW
CHUNK = 128
NCHUNK = B_PER_W // CHUNK


def _make_sc_kernel():
    mesh = plsc.VectorSubcoreMesh(core_axis_name="c", subcore_axis_name="s")

    @functools.partial(
        pl.kernel,
        mesh=mesh,
        compiler_params=pltpu.CompilerParams(use_tc_tiling_on_sc=False),
        out_type=jax.ShapeDtypeStruct((NW, LANES), jnp.float32),
        scratch_types=[
            pltpu.VMEM((NCHUNK, CHUNK), jnp.int32),
            pltpu.VMEM((NCHUNK, CHUNK, FEAT), jnp.float32),
            pltpu.VMEM((B_PER_W, FEAT), jnp.float32),
            pltpu.VMEM((LANES,), jnp.float32),
            pltpu.SemaphoreType.DMA,
        ],
    )
    def body(x_hbm, idx_hbm, table_hbm, out_hbm, idx_v, rows_v, x_v, acc_v, sem):
        wid = lax.axis_index("s") * NC + lax.axis_index("c")

        pltpu.sync_copy(idx_hbm.at[wid], idx_v)
        copies = [
            pltpu.async_copy(table_hbm.at[idx_v.at[k]], rows_v.at[k], sem)
            for k in range(NCHUNK)
        ]
        pltpu.sync_copy(x_hbm.at[wid], x_v)
        for cp in copies:
            cp.wait()

        def chunk_sum(k, acc):
            def row(r, acc):
                for c in range(FEAT // LANES):
                    xa = x_v[k * CHUNK + r, pl.ds(c * LANES, LANES)]
                    ga = rows_v[k, r, pl.ds(c * LANES, LANES)]
                    d = xa - ga
                    acc = acc + d * d
                return acc

            return lax.fori_loop(0, CHUNK, row, acc)

        acc = jnp.zeros((LANES,), jnp.float32)
        for k in range(NCHUNK):
            acc = chunk_sum(k, acc)
        acc_v[...] = acc
        pltpu.sync_copy(acc_v, out_hbm.at[wid])

    return body


_sc_loss_partials = _make_sc_kernel()


@jax.jit
def kernel(x, labels, centers):
    batch, feat = x.shape
    idx = labels.astype(jnp.int32).reshape(NW, NCHUNK, CHUNK)
    xr = x.reshape(NW, B_PER_W, feat)
    partials = _sc_loss_partials(xr, idx, centers)
    return jnp.sum(partials) / 2.0 / batch
